# Initial kernel scaffold; baseline (speedup 1.0000x reference)
import jax
import jax.numpy as jnp
from jax.experimental import pallas as pl


def kernel(x, mapping_table):
    raise NotImplementedError("write your pallas kernel here")


# speedup vs baseline: 578.0739x; 578.0739x over previous
import functools

import jax
import jax.numpy as jnp
from jax import lax
from jax.experimental import pallas as pl
from jax.experimental.pallas import tpu as pltpu
from jax.experimental.pallas import tpu_sc as plsc

B, C, H, W = 32, 3, 512, 512
NPIX = H * W
ROWS = B * C
NC, NS, L = 2, 16, 16
NW = NC * NS
ROWS_PER_W = ROWS // NW
CHUNK = 8192
NCHUNK = NPIX // CHUNK
UNROLL = 8
MAGIC = 12582912.0

_mesh = plsc.VectorSubcoreMesh(core_axis_name="c", subcore_axis_name="s")


@functools.partial(
    pl.kernel,
    mesh=_mesh,
    out_type=jax.ShapeDtypeStruct((ROWS, NPIX), jnp.float32),
    scratch_types=[
        pltpu.VMEM((256,), jnp.float32),
        pltpu.VMEM((CHUNK,), jnp.float32),
        pltpu.VMEM((CHUNK,), jnp.float32),
    ],
    compiler_params=pltpu.CompilerParams(needs_layout_passes=False),
)
def _lut_kernel(x_hbm, table_hbm, out_hbm, tab_v, in_v, out_v):
    wid = lax.axis_index("s") * NC + lax.axis_index("c")

    def do_row(r):
        row = wid * ROWS_PER_W + r
        pltpu.sync_copy(table_hbm.at[row], tab_v)

        def chunk_body(ci, carry):
            off = ci * CHUNK
            pltpu.sync_copy(x_hbm.at[row, pl.ds(off, CHUNK)], in_v)

            def vec_body(i, carry2):
                for j in range(UNROLL):
                    base = i * (L * UNROLL) + j * L
                    v = in_v[pl.ds(base, L)]
                    y = v * 255.0
                    y = jnp.minimum(jnp.maximum(y, 0.0), 255.0)
                    r_ = (y + MAGIC) - MAGIC
                    idx = r_.astype(jnp.int32)
                    out_v[pl.ds(base, L)] = plsc.load_gather(tab_v, [idx])
                return carry2

            lax.fori_loop(0, CHUNK // (L * UNROLL), vec_body, 0)
            pltpu.sync_copy(out_v, out_hbm.at[row, pl.ds(off, CHUNK)])
            return carry

        lax.fori_loop(0, NCHUNK, chunk_body, 0)

    for r in range(ROWS_PER_W):
        do_row(r)


def kernel(x, mapping_table):
    x2 = x.reshape(ROWS, NPIX)
    t2 = mapping_table.reshape(ROWS, 256)
    out = _lut_kernel(x2, t2)
    return out.reshape(B, C, H, W)


# speedup vs baseline: 856.3472x; 1.4814x over previous
import functools

import jax
import jax.numpy as jnp
from jax import lax
from jax.experimental import pallas as pl
from jax.experimental.pallas import tpu as pltpu
from jax.experimental.pallas import tpu_sc as plsc

B, C, H, W = 32, 3, 512, 512
NPIX = H * W
ROWS = B * C
NC, NS, L = 2, 16, 16
NW = NC * NS
ROWS_PER_W = ROWS // NW
SPAN = ROWS_PER_W * NPIX
CHUNK = 16384
CPR = NPIX // CHUNK
GTOT = ROWS_PER_W * CPR
UNROLL = 8
MAGIC = 12582912.0
BIAS = 0x4B400000

_mesh = plsc.VectorSubcoreMesh(core_axis_name="c", subcore_axis_name="s")


@functools.partial(
    pl.kernel,
    mesh=_mesh,
    out_type=jax.ShapeDtypeStruct((ROWS * NPIX,), jnp.float32),
    scratch_types=[
        pltpu.VMEM((ROWS_PER_W * 256,), jnp.float32),
        pltpu.VMEM((CHUNK,), jnp.float32),
        pltpu.VMEM((CHUNK,), jnp.float32),
        pltpu.VMEM((CHUNK,), jnp.float32),
        pltpu.VMEM((CHUNK,), jnp.float32),
        pltpu.SemaphoreType.DMA,
        pltpu.SemaphoreType.DMA,
        pltpu.SemaphoreType.DMA,
        pltpu.SemaphoreType.DMA,
    ],
    compiler_params=pltpu.CompilerParams(needs_layout_passes=False),
)
def _lut_kernel(x_hbm, table_hbm, out_hbm, tab_v, in0, in1, out0, out1,
                isem0, isem1, osem0, osem1):
    wid = lax.axis_index("s") * NC + lax.axis_index("c")
    base = wid * SPAN
    in_v = (in0, in1)
    out_v = (out0, out1)
    isem = (isem0, isem1)
    osem = (osem0, osem1)

    pltpu.sync_copy(table_hbm.at[pl.ds(wid * (ROWS_PER_W * 256),
                                       ROWS_PER_W * 256)], tab_v)

    def in_slice(g):
        return x_hbm.at[pl.ds(base + g * CHUNK, CHUNK)]

    def out_slice(g):
        return out_hbm.at[pl.ds(base + g * CHUNK, CHUNK)]

    for b in range(2):
        pltpu.async_copy(in_slice(b), in_v[b], isem[b])

    def pair_body(p, carry):
        for b in range(2):
            g = p * 2 + b
            pltpu.make_async_copy(in_slice(g), in_v[b], isem[b]).wait()

            @pl.when(g >= 2)
            def _wait_out():
                pltpu.make_async_copy(out_v[b], out_slice(g - 2),
                                      osem[b]).wait()

            c = BIAS - (g // CPR) * 256

            def vec_body(i, carry2):
                for j in range(UNROLL):
                    off = i * (L * UNROLL) + j * L
                    v = in_v[b][pl.ds(off, L)]
                    y = v * 255.0
                    y = jnp.minimum(jnp.maximum(y, 0.0), 255.0)
                    idx = plsc.bitcast(y + MAGIC, jnp.int32) - c
                    out_v[b][pl.ds(off, L)] = plsc.load_gather(tab_v, [idx])
                return carry2

            lax.fori_loop(0, CHUNK // (L * UNROLL), vec_body, 0)

            pltpu.async_copy(out_v[b], out_slice(g), osem[b])

            @pl.when(g + 2 < GTOT)
            def _next_in():
                pltpu.async_copy(in_slice(g + 2), in_v[b], isem[b])
        return carry

    lax.fori_loop(0, GTOT // 2, pair_body, 0)

    for b in range(2):
        pltpu.make_async_copy(out_v[b], out_slice(GTOT - 2 + b),
                              osem[b]).wait()


def kernel(x, mapping_table):
    x2 = x.reshape(ROWS * NPIX)
    t2 = mapping_table.reshape(ROWS * 256)
    out = _lut_kernel(x2, t2)
    return out.reshape(B, C, H, W)
